# baseline (speedup 1.0000x reference)
import jax
import jax.numpy as jnp
from jax import lax
from jax.experimental import pallas as pl
from jax.experimental.pallas import tpu as pltpu
from jax.experimental.pallas import tpu_sc as plsc

_NUM_HEADS = 16
_MAX_DIST = 12
_V = 1024
_NW = 32
_ROWS_PER_W = _V // _NW
_CR = 2
_NCHUNK = _ROWS_PER_W // _CR
_CHUNK_ELEMS = _CR * _V
_HEAD_STRIDE = _V * _V


def _sc_body(dist_hbm, tbl_hbm, out_hbm, tbl_v, dist0, dist1, out0, out1,
             sem_in0, sem_in1, sem_out0, sem_out1):
    wid = lax.axis_index("c") * 16 + lax.axis_index("s")
    base_row = wid * _ROWS_PER_W
    pltpu.sync_copy(tbl_hbm, tbl_v)
    rows = [tbl_v[pl.ds(h * 16, 16)] for h in range(_NUM_HEADS)]

    dist_bufs = (dist0, dist1)
    out_bufs = (out0, out1)
    sems_in = (sem_in0, sem_in1)
    sems_out = (sem_out0, sem_out1)

    def start_in(c, buf, sem):
        h = pltpu.make_async_copy(
            dist_hbm.at[pl.ds((base_row + c * _CR) * _V, _CHUNK_ELEMS)],
            buf, sem)
        h.start()
        return h

    in_handles = [start_in(0, dist0, sem_in0), None]
    out_handles = [None, None]
    for c in range(_NCHUNK):
        p = c % 2
        if c + 1 < _NCHUNK:
            in_handles[1 - p] = start_in(c + 1, dist_bufs[1 - p],
                                         sems_in[1 - p])
        in_handles[p].wait()
        if out_handles[p] is not None:
            for h in out_handles[p]:
                h.wait()
        dist_v = dist_bufs[p]
        out_v = out_bufs[p]

        def chunk_body(k, carry):
            idx = jnp.clip(dist_v[pl.ds(k * 16, 16)], 0, _MAX_DIST)
            for h in range(_NUM_HEADS):
                g = lax.gather(
                    rows[h], idx[:, None],
                    lax.GatherDimensionNumbers(
                        offset_dims=(), collapsed_slice_dims=(0,),
                        start_index_map=(0,)),
                    slice_sizes=(1,),
                    mode=lax.GatherScatterMode.PROMISE_IN_BOUNDS)
                out_v[pl.ds(h * _CHUNK_ELEMS + k * 16, 16)] = g
            return carry

        lax.fori_loop(0, _CHUNK_ELEMS // 16, chunk_body, 0)

        hs = []
        for h in range(_NUM_HEADS):
            hh = pltpu.make_async_copy(
                out_v.at[pl.ds(h * _CHUNK_ELEMS, _CHUNK_ELEMS)],
                out_hbm.at[pl.ds(h * _HEAD_STRIDE
                                 + (base_row + c * _CR) * _V, _CHUNK_ELEMS)],
                sems_out[p])
            hh.start()
            hs.append(hh)
        out_handles[p] = hs

    for p in range(2):
        if out_handles[p] is not None:
            for h in out_handles[p]:
                h.wait()


def kernel(dist_matrix, bias_table):
    dist_flat = dist_matrix.astype(jnp.int32).reshape(_V * _V)
    tbl_flat = jnp.pad(
        bias_table, ((0, 0), (0, 16 - (_MAX_DIST + 1)))).reshape(
            _NUM_HEADS * 16)
    mesh = plsc.VectorSubcoreMesh(core_axis_name="c", subcore_axis_name="s")
    run = pl.kernel(
        _sc_body,
        mesh=mesh,
        out_type=jax.ShapeDtypeStruct((_NUM_HEADS * _V * _V,), jnp.float32),
        scratch_types=[
            pltpu.VMEM((_NUM_HEADS * 16,), jnp.float32),
            pltpu.VMEM((_CHUNK_ELEMS,), jnp.int32),
            pltpu.VMEM((_CHUNK_ELEMS,), jnp.int32),
            pltpu.VMEM((_NUM_HEADS * _CHUNK_ELEMS,), jnp.float32),
            pltpu.VMEM((_NUM_HEADS * _CHUNK_ELEMS,), jnp.float32),
            pltpu.SemaphoreType.DMA,
            pltpu.SemaphoreType.DMA,
            pltpu.SemaphoreType.DMA,
            pltpu.SemaphoreType.DMA,
        ],
    )
    out_flat = run(dist_flat, tbl_flat)
    return out_flat.reshape(_NUM_HEADS, _V, _V)


# speedup vs baseline: 4.8297x; 4.8297x over previous
import jax
import jax.numpy as jnp
from jax.experimental import pallas as pl
from jax.experimental.pallas import tpu as pltpu

_NUM_HEADS = 16
_MAX_DIST = 12
_NB = _MAX_DIST + 1
_V = 1024
_ROWS_PER_STEP = 128
_GROUPS = _ROWS_PER_STEP // 8
_GRID = _V // _ROWS_PER_STEP
_NBUF = 3


def _bias_kernel(dist_ref, lhs_ref, out_ref, buf_ref, sem_ref):
    i = pl.program_id(0)
    b = jax.lax.rem(i, _NBUF)
    lhs = lhs_ref[...]

    @pl.when(i >= _NBUF)
    def _drain_for_reuse():
        pltpu.make_async_copy(
            buf_ref.at[b],
            out_ref.at[:, pl.ds((i - _NBUF) * _ROWS_PER_STEP,
                                _ROWS_PER_STEP), :],
            sem_ref.at[b]).wait()

    for g in range(_GROUPS):
        tile = jnp.clip(dist_ref[g * 8:(g + 1) * 8, :], 0, _MAX_DIST)
        iota = jax.lax.broadcasted_iota(jnp.int32, (_NB, 8, _V), 0)
        oh = (tile[None] == iota).astype(jnp.bfloat16)
        rhs = oh.reshape(_NB * 8, _V)
        res = jax.lax.dot(lhs, rhs, preferred_element_type=jnp.float32)
        buf_ref[b, :, g * 8:(g + 1) * 8, :] = res.reshape(_NUM_HEADS, 8, _V)

    pltpu.make_async_copy(
        buf_ref.at[b],
        out_ref.at[:, pl.ds(i * _ROWS_PER_STEP, _ROWS_PER_STEP), :],
        sem_ref.at[b]).start()

    @pl.when(i == _GRID - 1)
    def _drain_tail():
        for k in range(_NBUF):
            step = _GRID - _NBUF + k
            pltpu.make_async_copy(
                buf_ref.at[jax.lax.rem(jnp.int32(step), _NBUF)],
                out_ref.at[:, pl.ds(step * _ROWS_PER_STEP,
                                    _ROWS_PER_STEP), :],
                sem_ref.at[jax.lax.rem(jnp.int32(step), _NBUF)]).wait()


def kernel(dist_matrix, bias_table):
    eye8 = jnp.eye(8, dtype=jnp.float32)
    lhs = (bias_table[:, None, :, None] * eye8[None, :, None, :])
    lhs = lhs.reshape(_NUM_HEADS * 8, _NB * 8).astype(jnp.bfloat16)
    return pl.pallas_call(
        _bias_kernel,
        grid=(_GRID,),
        in_specs=[
            pl.BlockSpec((_ROWS_PER_STEP, _V), lambda i: (i, 0)),
            pl.BlockSpec((_NUM_HEADS * 8, _NB * 8), lambda i: (0, 0)),
        ],
        out_specs=pl.BlockSpec(memory_space=pl.ANY),
        out_shape=jax.ShapeDtypeStruct((_NUM_HEADS, _V, _V), jnp.float32),
        scratch_shapes=[
            pltpu.VMEM((_NBUF, _NUM_HEADS, _ROWS_PER_STEP, _V), jnp.float32),
            pltpu.SemaphoreType.DMA((_NBUF,)),
        ],
    )(dist_matrix.astype(jnp.int32), lhs)
